# baseline (speedup 1.0000x reference)
import functools

import jax
import jax.numpy as jnp
from jax import lax
from jax.experimental import pallas as pl
from jax.experimental.pallas import tpu as pltpu
from jax.experimental.pallas import tpu_sc as plsc

_NUM_FEATURES = 128
_FACTORS = 16
_NUSERS = 100000
_UPAD = 100000
_BATCH = 4096
_L = 16
_NW = 32
_BPW = _BATCH // _NW
_NCHUNK = _NUM_FEATURES // _L
_NGRP = _BPW // _L
_NHB = (_BPW * _FACTORS) // 128

_GDN = lax.GatherDimensionNumbers(
    offset_dims=(), collapsed_slice_dims=(0,), start_index_map=(0,))


def _dyngather(v, idx):
    return lax.gather(v, idx.reshape(_L, 1), _GDN, (1,),
                      mode=lax.GatherScatterMode.PROMISE_IN_BOUNDS)


def _lanebcast(v, k):
    return _dyngather(v, jnp.full((_L,), k, jnp.int32))


def _lanesum(v):
    lanes = lax.iota(jnp.int32, _L)
    for step in (8, 4, 2, 1):
        v = v + _dyngather(v, (lanes + step) % _L)
    return v


def _sc_body(user_h, item_h, fb_h, ib_h, ht_h, gt_h, k_h, c_h, out_h,
             uidx, iidx, idxh, hrows, krows, crows, ibv, gtv, fbv,
             xout, sem, sem2, sem3, sem4):
    c = lax.axis_index("c")
    s = lax.axis_index("s")
    wid = s * 2 + c
    base = wid * _BPW

    pltpu.sync_copy(user_h.at[pl.ds(base, _BPW)], uidx)
    pltpu.sync_copy(item_h.at[pl.ds(base, _BPW)], iidx)
    pltpu.sync_copy(gt_h, gtv)
    pltpu.sync_copy(fb_h, fbv)

    _NS = 4
    _HB = _BPW // _NS

    def _kc_copies(h, sm):
        hs = pl.ds(h * _HB, _HB)
        return (
            pltpu.make_async_copy(k_h.at[uidx.at[hs]], krows.at[hs], sm),
            pltpu.make_async_copy(c_h.at[iidx.at[hs]], crows.at[hs], sm),
            pltpu.make_async_copy(ib_h.at[iidx.at[hs]], ibv.at[hs], sm),
        )

    def _h_copies(h, sm):
        return [pltpu.make_async_copy(ht_h.at[idxh.at[j]], hrows.at[j], sm)
                for j in range(h * (_NHB // _NS), (h + 1) * (_NHB // _NS))]

    koff = lax.iota(jnp.int32, _L) * _UPAD
    sems = (sem, sem2, sem3, sem4)
    for h in range(_NS):
        for cp in _kc_copies(h, sems[h]):
            cp.start()
        for grp in range(h * (_NGRP // _NS), (h + 1) * (_NGRP // _NS)):
            uv = uidx[pl.ds(grp * _L, _L)]
            for lane in range(_L):
                b = grp * _L + lane
                vec = _lanebcast(uv, lane) + koff
                idxh[b // 8, pl.ds((b % 8) * _L, _L)] = vec
        for cp in _h_copies(h, sems[h]):
            cp.start()

    lanes = lax.iota(jnp.int32, _L)
    fbcs = [fbv[pl.ds(fc * _L, _L)] for fc in range(_NCHUNK)]

    def pair_body(bb, carry):
        b0 = bb * 2
        b1 = b0 + 1
        j = b0 // 8
        off = (b0 % 8) * _L
        hv0 = hrows[j, pl.ds(off, _L)]
        hv1 = hrows[j, pl.ds(off + _L, _L)]
        hb0 = [_lanebcast(hv0, k) for k in range(_FACTORS)]
        hb1 = [_lanebcast(hv1, k) for k in range(_FACTORS)]
        acc0 = jnp.zeros((_L,), jnp.float32)
        acc1 = jnp.zeros((_L,), jnp.float32)
        for fc in range(_NCHUNK):
            fsl = pl.ds(fc * _L, _L)
            z0 = [fbcs[fc], 0.0, 0.0, 0.0]
            z1 = [fbcs[fc], 0.0, 0.0, 0.0]
            for k in range(_FACTORS):
                g = gtv[k, fsl]
                z0[k % 4] = z0[k % 4] + hb0[k] * g
                z1[k % 4] = z1[k % 4] + hb1[k] * g
            zz0 = (z0[0] + z0[1]) + (z0[2] + z0[3])
            zz1 = (z1[0] + z1[1]) + (z1[2] + z1[3])
            w0 = krows[b0, fsl] * (crows[b0, fsl] - 1.0)
            w1 = krows[b1, fsl] * (crows[b1, fsl] - 1.0)
            acc0 = acc0 + w0 * zz0
            acc1 = acc1 + w1 * zz1
        s0 = _lanesum(acc0)
        s1 = _lanesum(acc1)
        lane0 = b0 % _L
        contrib = (jnp.where(lanes == lane0, s0, 0.0)
                   + jnp.where(lanes == lane0 + 1, s1, 0.0))
        gsl = pl.ds((b0 // _L) * _L, _L)
        xout[gsl] = xout[gsl] + contrib
        return carry

    for h in range(_NS):
        for cp in _kc_copies(h, sems[h]):
            cp.wait()
        for cp in _h_copies(h, sems[h]):
            cp.wait()
        for g in range(h * (_NGRP // _NS), (h + 1) * (_NGRP // _NS)):
            gs = pl.ds(g * _L, _L)
            xout[gs] = ibv[gs]
        lax.fori_loop(h * (_HB // 2), (h + 1) * (_HB // 2), pair_body, 0,
                      unroll=2)

    pltpu.sync_copy(xout, out_h.at[pl.ds(base, _BPW)])


@jax.jit
def _run(user, item, F_B, I_B, HTflat, GT, K, C):
    mesh = plsc.VectorSubcoreMesh(core_axis_name="c", subcore_axis_name="s")
    fn = pl.kernel(
        _sc_body,
        out_type=jax.ShapeDtypeStruct((_BATCH,), jnp.float32),
        mesh=mesh,
        compiler_params=pltpu.CompilerParams(use_tc_tiling_on_sc=False),
        scratch_types=[
            pltpu.VMEM((_BPW,), jnp.int32),
            pltpu.VMEM((_BPW,), jnp.int32),
            pltpu.VMEM((_NHB, 128), jnp.int32),
            pltpu.VMEM((_NHB, 128), jnp.float32),
            pltpu.VMEM((_BPW, _NUM_FEATURES), jnp.float32),
            pltpu.VMEM((_BPW, _NUM_FEATURES), jnp.float32),
            pltpu.VMEM((_BPW,), jnp.float32),
            pltpu.VMEM((_FACTORS, _NUM_FEATURES), jnp.float32),
            pltpu.VMEM((_NUM_FEATURES,), jnp.float32),
            pltpu.VMEM((_BPW,), jnp.float32),
            pltpu.SemaphoreType.DMA,
            pltpu.SemaphoreType.DMA,
            pltpu.SemaphoreType.DMA,
            pltpu.SemaphoreType.DMA,
        ],
    )
    return fn(user, item, F_B, I_B, HTflat, GT, K, C)


def kernel(user, item, F_B, I_B, H, G, K, C):
    ht_flat = H.T.reshape(-1)
    return _run(user.astype(jnp.int32), item.astype(jnp.int32),
                F_B, I_B, ht_flat, G.T, K, C)


# speedup vs baseline: 1.0457x; 1.0457x over previous
import functools

import jax
import jax.numpy as jnp
from jax import lax
from jax.experimental import pallas as pl
from jax.experimental.pallas import tpu as pltpu
from jax.experimental.pallas import tpu_sc as plsc

_NUM_FEATURES = 128
_FACTORS = 16
_NUSERS = 100000
_UPAD = 100000
_BATCH = 4096
_L = 16
_NW = 32
_BPW = _BATCH // _NW
_NCHUNK = _NUM_FEATURES // _L
_NGRP = _BPW // _L
_NHB = (_BPW * _FACTORS) // 128

_GDN = lax.GatherDimensionNumbers(
    offset_dims=(), collapsed_slice_dims=(0,), start_index_map=(0,))


def _dyngather(v, idx):
    return lax.gather(v, idx.reshape(_L, 1), _GDN, (1,),
                      mode=lax.GatherScatterMode.PROMISE_IN_BOUNDS)


def _lanebcast(v, k):
    return _dyngather(v, jnp.full((_L,), k, jnp.int32))


def _lanesum(v):
    lanes = lax.iota(jnp.int32, _L)
    for step in (8, 4, 2, 1):
        v = v + _dyngather(v, (lanes + step) % _L)
    return v


def _sc_body(user_h, item_h, fb_h, ib_h, ht_h, gt_h, k_h, c_h, out_h,
             uidx, iidx, idxh, hrows, krows, crows, ibv, gtv, fbv,
             xout, sem, sem2, sem3, sem4):
    c = lax.axis_index("c")
    s = lax.axis_index("s")
    wid = s * 2 + c
    base = wid * _BPW

    pltpu.sync_copy(user_h.at[pl.ds(base, _BPW)], uidx)
    pltpu.sync_copy(item_h.at[pl.ds(base, _BPW)], iidx)
    pltpu.sync_copy(gt_h, gtv)
    pltpu.sync_copy(fb_h, fbv)

    _NS = 4
    _HB = _BPW // _NS

    def _kc_copies(h, sm):
        hs = pl.ds(h * _HB, _HB)
        return (
            pltpu.make_async_copy(k_h.at[uidx.at[hs]], krows.at[hs], sm),
            pltpu.make_async_copy(c_h.at[iidx.at[hs]], crows.at[hs], sm),
            pltpu.make_async_copy(ib_h.at[iidx.at[hs]], ibv.at[hs], sm),
        )

    def _h_copies(h, sm):
        return [pltpu.make_async_copy(ht_h.at[idxh.at[j]], hrows.at[j], sm)
                for j in range(h * (_NHB // _NS), (h + 1) * (_NHB // _NS))]

    koff = lax.iota(jnp.int32, _L) * _UPAD
    sems = (sem, sem2, sem3, sem4)
    for h in range(_NS):
        for cp in _kc_copies(h, sems[h]):
            cp.start()
        for grp in range(h * (_NGRP // _NS), (h + 1) * (_NGRP // _NS)):
            uv = uidx[pl.ds(grp * _L, _L)]
            for lane in range(_L):
                b = grp * _L + lane
                vec = _lanebcast(uv, lane) + koff
                idxh[b // 8, pl.ds((b % 8) * _L, _L)] = vec
        for cp in _h_copies(h, sems[h]):
            cp.start()

    lanes = lax.iota(jnp.int32, _L)
    fbcs = [fbv[pl.ds(fc * _L, _L)] for fc in range(_NCHUNK)]

    def pair_body(bb, carry):
        b0 = bb * 2
        b1 = b0 + 1
        j = b0 // 8
        off = (b0 % 8) * _L
        hv0 = hrows[j, pl.ds(off, _L)]
        hv1 = hrows[j, pl.ds(off + _L, _L)]
        hb0 = [_lanebcast(hv0, k) for k in range(_FACTORS)]
        hb1 = [_lanebcast(hv1, k) for k in range(_FACTORS)]
        acc0 = jnp.zeros((_L,), jnp.float32)
        acc1 = jnp.zeros((_L,), jnp.float32)
        for fc in range(_NCHUNK):
            fsl = pl.ds(fc * _L, _L)
            z0 = [fbcs[fc], 0.0, 0.0, 0.0]
            z1 = [fbcs[fc], 0.0, 0.0, 0.0]
            for k in range(_FACTORS):
                g = gtv[k, fsl]
                z0[k % 4] = z0[k % 4] + hb0[k] * g
                z1[k % 4] = z1[k % 4] + hb1[k] * g
            zz0 = (z0[0] + z0[1]) + (z0[2] + z0[3])
            zz1 = (z1[0] + z1[1]) + (z1[2] + z1[3])
            w0 = krows[b0, fsl] * (crows[b0, fsl] - 1.0)
            w1 = krows[b1, fsl] * (crows[b1, fsl] - 1.0)
            acc0 = acc0 + w0 * zz0
            acc1 = acc1 + w1 * zz1
        s0 = _lanesum(acc0)
        s1 = _lanesum(acc1)
        lane0 = b0 % _L
        contrib = (jnp.where(lanes == lane0, s0, 0.0)
                   + jnp.where(lanes == lane0 + 1, s1, 0.0))
        gsl = pl.ds((b0 // _L) * _L, _L)
        xout[gsl] = xout[gsl] + contrib
        return carry

    for h in range(_NS):
        for cp in _kc_copies(h, sems[h]):
            cp.wait()
        for cp in _h_copies(h, sems[h]):
            cp.wait()
        for g in range(h * (_NGRP // _NS), (h + 1) * (_NGRP // _NS)):
            gs = pl.ds(g * _L, _L)
            xout[gs] = ibv[gs]
        lax.fori_loop(h * (_HB // 2), (h + 1) * (_HB // 2), pair_body, 0)

    pltpu.sync_copy(xout, out_h.at[pl.ds(base, _BPW)])


@jax.jit
def _run(user, item, F_B, I_B, HTflat, GT, K, C):
    mesh = plsc.VectorSubcoreMesh(core_axis_name="c", subcore_axis_name="s")
    fn = pl.kernel(
        _sc_body,
        out_type=jax.ShapeDtypeStruct((_BATCH,), jnp.float32),
        mesh=mesh,
        compiler_params=pltpu.CompilerParams(use_tc_tiling_on_sc=False),
        scratch_types=[
            pltpu.VMEM((_BPW,), jnp.int32),
            pltpu.VMEM((_BPW,), jnp.int32),
            pltpu.VMEM((_NHB, 128), jnp.int32),
            pltpu.VMEM((_NHB, 128), jnp.float32),
            pltpu.VMEM((_BPW, _NUM_FEATURES), jnp.float32),
            pltpu.VMEM((_BPW, _NUM_FEATURES), jnp.float32),
            pltpu.VMEM((_BPW,), jnp.float32),
            pltpu.VMEM((_FACTORS, _NUM_FEATURES), jnp.float32),
            pltpu.VMEM((_NUM_FEATURES,), jnp.float32),
            pltpu.VMEM((_BPW,), jnp.float32),
            pltpu.SemaphoreType.DMA,
            pltpu.SemaphoreType.DMA,
            pltpu.SemaphoreType.DMA,
            pltpu.SemaphoreType.DMA,
        ],
    )
    return fn(user, item, F_B, I_B, HTflat, GT, K, C)


def kernel(user, item, F_B, I_B, H, G, K, C):
    ht_flat = H.T.reshape(-1)
    return _run(user.astype(jnp.int32), item.astype(jnp.int32),
                F_B, I_B, ht_flat, G.T, K, C)


# speedup vs baseline: 1.0847x; 1.0373x over previous
import functools

import jax
import jax.numpy as jnp
from jax import lax
from jax.experimental import pallas as pl
from jax.experimental.pallas import tpu as pltpu
from jax.experimental.pallas import tpu_sc as plsc

_NUM_FEATURES = 128
_FACTORS = 16
_NUSERS = 100000
_UPAD = 100000
_BATCH = 4096
_L = 16
_NW = 32
_BPW = _BATCH // _NW
_NCHUNK = _NUM_FEATURES // _L
_NGRP = _BPW // _L
_NHB = (_BPW * _FACTORS) // 128

_GDN = lax.GatherDimensionNumbers(
    offset_dims=(), collapsed_slice_dims=(0,), start_index_map=(0,))


def _dyngather(v, idx):
    return lax.gather(v, idx.reshape(_L, 1), _GDN, (1,),
                      mode=lax.GatherScatterMode.PROMISE_IN_BOUNDS)


def _lanebcast(v, k):
    return _dyngather(v, jnp.full((_L,), k, jnp.int32))


def _lanesum(v):
    lanes = lax.iota(jnp.int32, _L)
    for step in (8, 4, 2, 1):
        v = v + _dyngather(v, (lanes + step) % _L)
    return v


def _sc_body(user_h, item_h, fb_h, ib_h, ht_h, gt_h, k_h, c_h, out_h,
             uidx, iidx, idxh, hrows, krows, crows, ibv, gtv, fbv,
             xout, sem, sem2, sem3, sem4):
    c = lax.axis_index("c")
    s = lax.axis_index("s")
    wid = s * 2 + c
    base = wid * _BPW

    pltpu.sync_copy(user_h.at[pl.ds(base, _BPW)], uidx)
    pltpu.sync_copy(item_h.at[pl.ds(base, _BPW)], iidx)
    pltpu.sync_copy(gt_h, gtv)
    pltpu.sync_copy(fb_h, fbv)

    _SB = (0, 32, _BPW)
    _NS = len(_SB) - 1

    def _kc_copies(h, sm):
        hs = pl.ds(_SB[h], _SB[h + 1] - _SB[h])
        return (
            pltpu.make_async_copy(k_h.at[uidx.at[hs]], krows.at[hs], sm),
            pltpu.make_async_copy(c_h.at[iidx.at[hs]], crows.at[hs], sm),
            pltpu.make_async_copy(ib_h.at[iidx.at[hs]], ibv.at[hs], sm),
        )

    def _h_copies(h, sm):
        return [pltpu.make_async_copy(ht_h.at[idxh.at[j]], hrows.at[j], sm)
                for j in range(_SB[h] // 8, _SB[h + 1] // 8)]

    koff = lax.iota(jnp.int32, _L) * _UPAD
    sems = (sem, sem2, sem3, sem4)
    for h in range(_NS):
        for cp in _kc_copies(h, sems[h]):
            cp.start()
        for grp in range(_SB[h] // _L, _SB[h + 1] // _L):
            uv = uidx[pl.ds(grp * _L, _L)]
            for lane in range(_L):
                b = grp * _L + lane
                vec = _lanebcast(uv, lane) + koff
                idxh[b // 8, pl.ds((b % 8) * _L, _L)] = vec
        for cp in _h_copies(h, sems[h]):
            cp.start()

    lanes = lax.iota(jnp.int32, _L)
    fbcs = [fbv[pl.ds(fc * _L, _L)] for fc in range(_NCHUNK)]

    def pair_body(bb, carry):
        b0 = bb * 2
        b1 = b0 + 1
        j = b0 // 8
        off = (b0 % 8) * _L
        hv0 = hrows[j, pl.ds(off, _L)]
        hv1 = hrows[j, pl.ds(off + _L, _L)]
        hb0 = [_lanebcast(hv0, k) for k in range(_FACTORS)]
        hb1 = [_lanebcast(hv1, k) for k in range(_FACTORS)]
        acc0 = jnp.zeros((_L,), jnp.float32)
        acc1 = jnp.zeros((_L,), jnp.float32)
        for fc in range(_NCHUNK):
            fsl = pl.ds(fc * _L, _L)
            z0 = [fbcs[fc], 0.0, 0.0, 0.0]
            z1 = [fbcs[fc], 0.0, 0.0, 0.0]
            for k in range(_FACTORS):
                g = gtv[k, fsl]
                z0[k % 4] = z0[k % 4] + hb0[k] * g
                z1[k % 4] = z1[k % 4] + hb1[k] * g
            zz0 = (z0[0] + z0[1]) + (z0[2] + z0[3])
            zz1 = (z1[0] + z1[1]) + (z1[2] + z1[3])
            w0 = krows[b0, fsl] * (crows[b0, fsl] - 1.0)
            w1 = krows[b1, fsl] * (crows[b1, fsl] - 1.0)
            acc0 = acc0 + w0 * zz0
            acc1 = acc1 + w1 * zz1
        s0 = _lanesum(acc0)
        s1 = _lanesum(acc1)
        lane0 = b0 % _L
        contrib = (jnp.where(lanes == lane0, s0, 0.0)
                   + jnp.where(lanes == lane0 + 1, s1, 0.0))
        gsl = pl.ds((b0 // _L) * _L, _L)
        xout[gsl] = xout[gsl] + contrib
        return carry

    for h in range(_NS):
        for cp in _kc_copies(h, sems[h]):
            cp.wait()
        for cp in _h_copies(h, sems[h]):
            cp.wait()
        for g in range(_SB[h] // _L, _SB[h + 1] // _L):
            gs = pl.ds(g * _L, _L)
            xout[gs] = ibv[gs]
        lax.fori_loop(_SB[h] // 2, _SB[h + 1] // 2, pair_body, 0)

    pltpu.sync_copy(xout, out_h.at[pl.ds(base, _BPW)])


@jax.jit
def _run(user, item, F_B, I_B, HTflat, GT, K, C):
    mesh = plsc.VectorSubcoreMesh(core_axis_name="c", subcore_axis_name="s")
    fn = pl.kernel(
        _sc_body,
        out_type=jax.ShapeDtypeStruct((_BATCH,), jnp.float32),
        mesh=mesh,
        compiler_params=pltpu.CompilerParams(use_tc_tiling_on_sc=False),
        scratch_types=[
            pltpu.VMEM((_BPW,), jnp.int32),
            pltpu.VMEM((_BPW,), jnp.int32),
            pltpu.VMEM((_NHB, 128), jnp.int32),
            pltpu.VMEM((_NHB, 128), jnp.float32),
            pltpu.VMEM((_BPW, _NUM_FEATURES), jnp.float32),
            pltpu.VMEM((_BPW, _NUM_FEATURES), jnp.float32),
            pltpu.VMEM((_BPW,), jnp.float32),
            pltpu.VMEM((_FACTORS, _NUM_FEATURES), jnp.float32),
            pltpu.VMEM((_NUM_FEATURES,), jnp.float32),
            pltpu.VMEM((_BPW,), jnp.float32),
            pltpu.SemaphoreType.DMA,
            pltpu.SemaphoreType.DMA,
            pltpu.SemaphoreType.DMA,
            pltpu.SemaphoreType.DMA,
        ],
    )
    return fn(user, item, F_B, I_B, HTflat, GT, K, C)


def kernel(user, item, F_B, I_B, H, G, K, C):
    ht_flat = H.T.reshape(-1)
    return _run(user.astype(jnp.int32), item.astype(jnp.int32),
                F_B, I_B, ht_flat, G.T, K, C)
